# baseline (speedup 1.0000x reference)
import functools

import jax
import jax.numpy as jnp
from jax import lax
from jax.experimental import pallas as pl
from jax.experimental.pallas import tpu as pltpu
from jax.experimental.pallas import tpu_sc as plsc

N, E, DF, DE, DG, H = 10000, 320000, 128, 16, 32, 128

NW = 32
EPW = E // NW
C = 40
NCHUNK = EPW // C
NSTRIPE = 10
SRW = N // NSTRIPE
F32 = jnp.float32


def _pq_body(nodes_ref, ws_ref, wd_ref, p_ref, q_ref):
    x = nodes_ref[...]
    p_ref[...] = jnp.dot(x, ws_ref[...], preferred_element_type=F32)
    q_ref[...] = jnp.dot(x, wd_ref[...], preferred_element_type=F32)


def _project_nodes(nodes, ws, wd):
    blk = 1000
    grid = N // blk
    return pl.pallas_call(
        _pq_body,
        grid=(grid,),
        in_specs=[
            pl.BlockSpec((blk, DF), lambda i: (i, 0)),
            pl.BlockSpec((DF, H), lambda i: (0, 0)),
            pl.BlockSpec((DF, H), lambda i: (0, 0)),
        ],
        out_specs=[
            pl.BlockSpec((blk, H), lambda i: (i, 0)),
            pl.BlockSpec((blk, H), lambda i: (i, 0)),
        ],
        out_shape=[
            jax.ShapeDtypeStruct((N, H), F32),
            jax.ShapeDtypeStruct((N, H), F32),
        ],
    )(nodes, ws, wd)


def _xe_body(edges_ref, w_ref, c_ref, xe_ref):
    xe_ref[...] = (
        jnp.dot(edges_ref[...], w_ref[...], preferred_element_type=F32)
        + c_ref[...]
    )


def _project_edges(edges, we1, ce):
    blk = 8000
    grid = E // blk
    return pl.pallas_call(
        _xe_body,
        grid=(grid,),
        in_specs=[
            pl.BlockSpec((blk, DE), lambda i: (i, 0)),
            pl.BlockSpec((DE, H), lambda i: (0, 0)),
            pl.BlockSpec((1, H), lambda i: (0, 0)),
        ],
        out_specs=pl.BlockSpec((blk, H), lambda i: (i, 0)),
        out_shape=jax.ShapeDtypeStruct((E, H), F32),
    )(edges, we1, ce)


def _sc_edge_body(p_hbm, q_hbm, xe_hbm, s_hbm, r_hbm,
                  enew_hbm, agg_hbm,
                  sb0, sb1, sb2, sb3, rb0, rb1, rb2, rb3,
                  pb0, pb1, qb0, qb1, xb0, xb1, ob0, ob1, agg_sh,
                  sem_is0, sem_is1, sem_is2, sem_is3,
                  sem_ir0, sem_ir1, sem_ir2, sem_ir3,
                  sem_gp0, sem_gp1, sem_gq0, sem_gq1, sem_gx0, sem_gx1,
                  sem_e0, sem_e1, sem_a0, sem_a1):
    cid = lax.axis_index("c")
    sid = lax.axis_index("s")
    wid = cid * 16 + sid
    base = wid * EPW
    sbufs = (sb0, sb1, sb2, sb3)
    rbufs = (rb0, rb1, rb2, rb3)
    pbuf = (pb0, pb1)
    qbuf = (qb0, qb1)
    xbuf = (xb0, xb1)
    obuf = (ob0, ob1)
    sem_is = (sem_is0, sem_is1, sem_is2, sem_is3)
    sem_ir = (sem_ir0, sem_ir1, sem_ir2, sem_ir3)
    sem_gp = (sem_gp0, sem_gp1)
    sem_gq = (sem_gq0, sem_gq1)
    sem_gx = (sem_gx0, sem_gx1)
    sem_e = (sem_e0, sem_e1)
    sem_a = (sem_a0, sem_a1)

    zeros = jnp.zeros((16,), F32)

    def zrow(r, _):
        for c8 in range(H // 16):
            xb0[r, pl.ds(c8 * 16, 16)] = zeros
        return 0

    lax.fori_loop(0, C, zrow, 0)

    @pl.when(sid < NSTRIPE)
    def _zero_stripe():
        for k in range(SRW // C):
            pltpu.sync_copy(xb0, agg_sh.at[pl.ds(sid * SRW + k * C, C)])

    plsc.subcore_barrier()

    def idx_descs(c, slot):
        sl = pl.ds(base + c * C, C)
        return (
            pltpu.make_async_copy(s_hbm.at[sl], sbufs[slot], sem_is[slot]),
            pltpu.make_async_copy(r_hbm.at[sl], rbufs[slot], sem_ir[slot]),
        )

    def gather_descs(c, b, slot):
        return (
            pltpu.make_async_copy(p_hbm.at[sbufs[slot]], pbuf[b], sem_gp[b]),
            pltpu.make_async_copy(q_hbm.at[rbufs[slot]], qbuf[b], sem_gq[b]),
            pltpu.make_async_copy(
                xe_hbm.at[pl.ds(base + c * C, C)], xbuf[b], sem_gx[b]
            ),
        )

    def store_descs(c, b, slot):
        return (
            pltpu.make_async_copy(
                obuf[b], enew_hbm.at[pl.ds(base + c * C, C)], sem_e[b]
            ),
            pltpu.make_async_copy(
                obuf[b], agg_sh.at[rbufs[slot]], sem_a[b]
            ),
        )

    def fire_stores(c, b, slot):
        pltpu.async_copy(obuf[b], enew_hbm.at[pl.ds(base + c * C, C)],
                         sem_e[b])
        pltpu.async_copy(obuf[b], agg_sh.at[rbufs[slot]], sem_a[b],
                         add=True)

    def compute(b):
        def row(r4, _):
            for rr in range(4):
                r = r4 * 4 + rr
                for c8 in range(H // 16):
                    sl = pl.ds(c8 * 16, 16)
                    v = xbuf[b][r, sl] + pbuf[b][r, sl] + qbuf[b][r, sl]
                    obuf[b][r, sl] = jnp.maximum(v, 0.0)
            return 0

        lax.fori_loop(0, C // 4, row, 0)

    def body(c, b, slot_c, slot_n1, slot_n2, *, wait_st=True, fire_ix=True,
             wait_ix=True, fire_g=True):
        if wait_st:
            for d in store_descs(c - 2, b, slot_n2):
                d.wait()
        for d in gather_descs(c, b, slot_c):
            d.wait()
        if fire_ix:
            for d in idx_descs(c + 2, slot_n2):
                d.start()
        if wait_ix:
            for d in idx_descs(c + 1, slot_n1):
                d.wait()
        if fire_g:
            for d in gather_descs(c + 1, 1 - b, slot_n1):
                d.start()
        compute(b)
        fire_stores(c, b, slot_c)

    for d in idx_descs(0, 0) + idx_descs(1, 1):
        d.start()
    for d in idx_descs(0, 0):
        d.wait()
    for d in gather_descs(0, 0, 0):
        d.start()
    body(0, 0, 0, 1, 2, wait_st=False)
    body(1, 1, 1, 2, 3, wait_st=False)

    def quad(c0, _):
        for j in range(4):
            body(c0 + j, j % 2, (2 + j) % 4, (3 + j) % 4, j % 4)
        return 0

    lax.fori_loop(0, (NCHUNK - 6) // 4, lambda i, a: quad(2 + i * 4, a), 0)

    body(NCHUNK - 4, 0, 2, 3, 0)
    body(NCHUNK - 3, 1, 3, 0, 1)
    body(NCHUNK - 2, 0, 0, 1, 2, fire_ix=False)
    body(NCHUNK - 1, 1, 1, 2, 3, fire_ix=False, wait_ix=False, fire_g=False)
    for d in store_descs(NCHUNK - 2, 0, 0):
        d.wait()
    for d in store_descs(NCHUNK - 1, 1, 1):
        d.wait()

    plsc.subcore_barrier()

    @pl.when(sid < NSTRIPE)
    def _publish():
        for k in range(SRW // C):
            off = sid * SRW + k * C
            pltpu.sync_copy(agg_sh.at[pl.ds(off, C)], xb0)
            pltpu.sync_copy(xb0, agg_hbm.at[pl.ds(cid * N + off, C)])


_sc_edge = functools.partial(
    pl.kernel,
    out_type=[
        jax.ShapeDtypeStruct((E, H), F32),
        jax.ShapeDtypeStruct((2 * N, H), F32),
    ],
    mesh=plsc.VectorSubcoreMesh(core_axis_name="c", subcore_axis_name="s"),
    scratch_types=(
        [pltpu.VMEM((C,), jnp.int32)] * 8
        + [pltpu.VMEM((C, H), F32)] * 8
        + [pltpu.VMEM_SHARED((N, H), F32)]
        + [pltpu.SemaphoreType.DMA] * 18
    ),
)(_sc_edge_body)


def _node_body(nodes_ref, a0_ref, a1_ref, wn1_ref, wn2_ref, cn_ref,
               wg1_ref, wg2_ref, cg_ref, n_ref, g_ref, sn_ref, sa_ref):
    i = pl.program_id(0)

    @pl.when(i == 0)
    def _init():
        sn_ref[...] = jnp.zeros_like(sn_ref)
        sa_ref[...] = jnp.zeros_like(sa_ref)

    agg = a0_ref[...] + a1_ref[...]
    nnew = jnp.maximum(
        jnp.dot(nodes_ref[...], wn1_ref[...], preferred_element_type=F32)
        + jnp.dot(agg, wn2_ref[...], preferred_element_type=F32)
        + cn_ref[...],
        0.0,
    )
    n_ref[...] = nnew
    sn_ref[...] += jnp.sum(nnew, axis=0, keepdims=True)
    sa_ref[...] += jnp.sum(agg, axis=0, keepdims=True)

    @pl.when(i == pl.num_programs(0) - 1)
    def _fin():
        mean_n = sn_ref[...] / jnp.float32(N)
        mean_e = sa_ref[...] / jnp.float32(E)
        g_ref[...] = (
            jnp.dot(mean_n, wg1_ref[...], preferred_element_type=F32)
            + jnp.dot(mean_e, wg2_ref[...], preferred_element_type=F32)
            + cg_ref[...]
        )


def _node_global(nodes, agg2, wn1, wn2, cn, wg1, wg2, cg):
    blk = 1000
    grid = N // blk
    return pl.pallas_call(
        _node_body,
        grid=(grid,),
        in_specs=[
            pl.BlockSpec((blk, DF), lambda i: (i, 0)),
            pl.BlockSpec((blk, H), lambda i: (i, 0)),
            pl.BlockSpec((blk, H), lambda i: (i + grid, 0)),
            pl.BlockSpec((DF, H), lambda i: (0, 0)),
            pl.BlockSpec((H, H), lambda i: (0, 0)),
            pl.BlockSpec((1, H), lambda i: (0, 0)),
            pl.BlockSpec((H, DG), lambda i: (0, 0)),
            pl.BlockSpec((H, DG), lambda i: (0, 0)),
            pl.BlockSpec((1, DG), lambda i: (0, 0)),
        ],
        out_specs=[
            pl.BlockSpec((blk, H), lambda i: (i, 0)),
            pl.BlockSpec((1, DG), lambda i: (0, 0)),
        ],
        out_shape=[
            jax.ShapeDtypeStruct((N, H), F32),
            jax.ShapeDtypeStruct((1, DG), F32),
        ],
        scratch_shapes=[
            pltpu.VMEM((1, H), F32),
            pltpu.VMEM((1, H), F32),
        ],
    )(nodes, agg2, agg2, wn1, wn2, cn, wg1, wg2, cg)


def kernel(nodes, edges, globals_, We_w, We_b, Wn_w, Wn_b, Wg_w, Wg_b,
           senders, receivers):
    senders = senders.astype(jnp.int32)
    receivers = receivers.astype(jnp.int32)

    we1 = We_w[:DE]
    ws = We_w[DE:DE + DF]
    wd = We_w[DE + DF:DE + 2 * DF]
    ce = (We_b + globals_ @ We_w[DE + 2 * DF:]).reshape(1, H)

    wn1 = Wn_w[:DF]
    wn2 = Wn_w[DF:DF + H]
    cn = (Wn_b + globals_ @ Wn_w[DF + H:]).reshape(1, H)

    wg1 = Wg_w[:H]
    wg2 = Wg_w[H:2 * H]
    cg = (Wg_b + globals_ @ Wg_w[2 * H:]).reshape(1, DG)

    p, q = _project_nodes(nodes, ws, wd)
    xe = _project_edges(edges, we1, ce)
    e_new, agg2 = _sc_edge(p, q, xe, senders, receivers)
    n_new, g_new = _node_global(nodes, agg2, wn1, wn2, cn, wg1, wg2, cg)
    return n_new, e_new, g_new


# speedup vs baseline: 1.0039x; 1.0039x over previous
import functools

import jax
import jax.numpy as jnp
from jax import lax
from jax.experimental import pallas as pl
from jax.experimental.pallas import tpu as pltpu
from jax.experimental.pallas import tpu_sc as plsc

N, E, DF, DE, DG, H = 10000, 320000, 128, 16, 32, 128

NW = 32
EPW = E // NW
C = 40
NCHUNK = EPW // C
NSTRIPE = 10
SRW = N // NSTRIPE
F32 = jnp.float32


def _pq_body(nodes_ref, ws_ref, wd_ref, p_ref, q_ref):
    x = nodes_ref[...]
    p_ref[...] = jnp.dot(x, ws_ref[...], preferred_element_type=F32)
    q_ref[...] = jnp.dot(x, wd_ref[...], preferred_element_type=F32)


def _project_nodes(nodes, ws, wd):
    blk = 1000
    grid = N // blk
    return pl.pallas_call(
        _pq_body,
        grid=(grid,),
        in_specs=[
            pl.BlockSpec((blk, DF), lambda i: (i, 0)),
            pl.BlockSpec((DF, H), lambda i: (0, 0)),
            pl.BlockSpec((DF, H), lambda i: (0, 0)),
        ],
        out_specs=[
            pl.BlockSpec((blk, H), lambda i: (i, 0)),
            pl.BlockSpec((blk, H), lambda i: (i, 0)),
        ],
        out_shape=[
            jax.ShapeDtypeStruct((N, H), F32),
            jax.ShapeDtypeStruct((N, H), F32),
        ],
    )(nodes, ws, wd)


def _xe_body(edges_ref, w_ref, c_ref, xe_ref):
    xe_ref[...] = (
        jnp.dot(edges_ref[...], w_ref[...], preferred_element_type=F32)
        + c_ref[...]
    )


def _project_edges(edges, we1, ce):
    blk = 8000
    grid = E // blk
    return pl.pallas_call(
        _xe_body,
        grid=(grid,),
        in_specs=[
            pl.BlockSpec((blk, DE), lambda i: (i, 0)),
            pl.BlockSpec((DE, H), lambda i: (0, 0)),
            pl.BlockSpec((1, H), lambda i: (0, 0)),
        ],
        out_specs=pl.BlockSpec((blk, H), lambda i: (i, 0)),
        out_shape=jax.ShapeDtypeStruct((E, H), F32),
    )(edges, we1, ce)


def _sc_edge_body(p_hbm, q_hbm, xe_hbm, s_hbm, r_hbm,
                  enew_hbm, agg_hbm,
                  sb0, sb1, sb2, sb3, rb0, rb1, rb2, rb3,
                  pb0, pb1, qb0, qb1, xb0, xb1, ob0, ob1, agg_sh,
                  sem_is0, sem_is1, sem_is2, sem_is3,
                  sem_ir0, sem_ir1, sem_ir2, sem_ir3,
                  sem_gp0, sem_gp1, sem_gq0, sem_gq1, sem_gx0, sem_gx1,
                  sem_e0, sem_e1, sem_a0, sem_a1):
    cid = lax.axis_index("c")
    sid = lax.axis_index("s")
    wid = cid * 16 + sid
    base = wid * EPW
    sbufs = (sb0, sb1, sb2, sb3)
    rbufs = (rb0, rb1, rb2, rb3)
    pbuf = (pb0, pb1)
    qbuf = (qb0, qb1)
    xbuf = (xb0, xb1)
    obuf = (ob0, ob1)
    sem_is = (sem_is0, sem_is1, sem_is2, sem_is3)
    sem_ir = (sem_ir0, sem_ir1, sem_ir2, sem_ir3)
    sem_gp = (sem_gp0, sem_gp1)
    sem_gq = (sem_gq0, sem_gq1)
    sem_gx = (sem_gx0, sem_gx1)
    sem_e = (sem_e0, sem_e1)
    sem_a = (sem_a0, sem_a1)

    zeros = jnp.zeros((16,), F32)

    def zrow(r, _):
        for c8 in range(H // 16):
            xb0[r, pl.ds(c8 * 16, 16)] = zeros
        return 0

    lax.fori_loop(0, C, zrow, 0)

    @pl.when(sid < NSTRIPE)
    def _zero_stripe():
        for k in range(SRW // C):
            pltpu.sync_copy(xb0, agg_sh.at[pl.ds(sid * SRW + k * C, C)])

    plsc.subcore_barrier()

    def idx_descs(c, slot):
        sl = pl.ds(base + c * C, C)
        return (
            pltpu.make_async_copy(s_hbm.at[sl], sbufs[slot], sem_is[slot]),
            pltpu.make_async_copy(r_hbm.at[sl], rbufs[slot], sem_ir[slot]),
        )

    def gather_descs(c, b, slot):
        return (
            pltpu.make_async_copy(p_hbm.at[sbufs[slot]], pbuf[b], sem_gp[b]),
            pltpu.make_async_copy(q_hbm.at[rbufs[slot]], qbuf[b], sem_gq[b]),
            pltpu.make_async_copy(
                xe_hbm.at[pl.ds(base + c * C, C)], xbuf[b], sem_gx[b]
            ),
        )

    def store_descs(c, b, slot):
        return (
            pltpu.make_async_copy(
                obuf[b], enew_hbm.at[pl.ds(base + c * C, C)], sem_e[b]
            ),
            pltpu.make_async_copy(
                obuf[b], agg_sh.at[rbufs[slot]], sem_a[b]
            ),
        )

    def fire_stores(c, b, slot):
        pltpu.async_copy(obuf[b], enew_hbm.at[pl.ds(base + c * C, C)],
                         sem_e[b])
        pltpu.async_copy(obuf[b], agg_sh.at[rbufs[slot]], sem_a[b],
                         add=True)

    def compute(b):
        def row(r2, _):
            for rr in range(2):
                r = r2 * 2 + rr
                for c8 in range(H // 16):
                    sl = pl.ds(c8 * 16, 16)
                    v = xbuf[b][r, sl] + pbuf[b][r, sl] + qbuf[b][r, sl]
                    obuf[b][r, sl] = jnp.maximum(v, 0.0)
            return 0

        lax.fori_loop(0, C // 2, row, 0)

    def body(c, b, slot_c, slot_n1, slot_n2, *, wait_st=True, fire_ix=True,
             wait_ix=True, fire_g=True):
        if wait_st:
            for d in store_descs(c - 2, b, slot_n2):
                d.wait()
        for d in gather_descs(c, b, slot_c):
            d.wait()
        if fire_ix:
            for d in idx_descs(c + 2, slot_n2):
                d.start()
        if wait_ix:
            for d in idx_descs(c + 1, slot_n1):
                d.wait()
        if fire_g:
            for d in gather_descs(c + 1, 1 - b, slot_n1):
                d.start()
        compute(b)
        fire_stores(c, b, slot_c)

    for d in idx_descs(0, 0) + idx_descs(1, 1):
        d.start()
    for d in idx_descs(0, 0):
        d.wait()
    for d in gather_descs(0, 0, 0):
        d.start()
    body(0, 0, 0, 1, 2, wait_st=False)
    body(1, 1, 1, 2, 3, wait_st=False)

    def quad(c0, _):
        for j in range(4):
            body(c0 + j, j % 2, (2 + j) % 4, (3 + j) % 4, j % 4)
        return 0

    lax.fori_loop(0, (NCHUNK - 6) // 4, lambda i, a: quad(2 + i * 4, a), 0)

    body(NCHUNK - 4, 0, 2, 3, 0)
    body(NCHUNK - 3, 1, 3, 0, 1)
    body(NCHUNK - 2, 0, 0, 1, 2, fire_ix=False)
    body(NCHUNK - 1, 1, 1, 2, 3, fire_ix=False, wait_ix=False, fire_g=False)
    for d in store_descs(NCHUNK - 2, 0, 0):
        d.wait()
    for d in store_descs(NCHUNK - 1, 1, 1):
        d.wait()

    plsc.subcore_barrier()

    @pl.when(sid < NSTRIPE)
    def _publish():
        for k in range(SRW // C):
            off = sid * SRW + k * C
            pltpu.sync_copy(agg_sh.at[pl.ds(off, C)], xb0)
            pltpu.sync_copy(xb0, agg_hbm.at[pl.ds(cid * N + off, C)])


_sc_edge = functools.partial(
    pl.kernel,
    out_type=[
        jax.ShapeDtypeStruct((E, H), F32),
        jax.ShapeDtypeStruct((2 * N, H), F32),
    ],
    mesh=plsc.VectorSubcoreMesh(core_axis_name="c", subcore_axis_name="s"),
    scratch_types=(
        [pltpu.VMEM((C,), jnp.int32)] * 8
        + [pltpu.VMEM((C, H), F32)] * 8
        + [pltpu.VMEM_SHARED((N, H), F32)]
        + [pltpu.SemaphoreType.DMA] * 18
    ),
)(_sc_edge_body)


def _node_body(nodes_ref, a0_ref, a1_ref, wn1_ref, wn2_ref, cn_ref,
               wg1_ref, wg2_ref, cg_ref, n_ref, g_ref, sn_ref, sa_ref):
    i = pl.program_id(0)

    @pl.when(i == 0)
    def _init():
        sn_ref[...] = jnp.zeros_like(sn_ref)
        sa_ref[...] = jnp.zeros_like(sa_ref)

    agg = a0_ref[...] + a1_ref[...]
    nnew = jnp.maximum(
        jnp.dot(nodes_ref[...], wn1_ref[...], preferred_element_type=F32)
        + jnp.dot(agg, wn2_ref[...], preferred_element_type=F32)
        + cn_ref[...],
        0.0,
    )
    n_ref[...] = nnew
    sn_ref[...] += jnp.sum(nnew, axis=0, keepdims=True)
    sa_ref[...] += jnp.sum(agg, axis=0, keepdims=True)

    @pl.when(i == pl.num_programs(0) - 1)
    def _fin():
        mean_n = sn_ref[...] / jnp.float32(N)
        mean_e = sa_ref[...] / jnp.float32(E)
        g_ref[...] = (
            jnp.dot(mean_n, wg1_ref[...], preferred_element_type=F32)
            + jnp.dot(mean_e, wg2_ref[...], preferred_element_type=F32)
            + cg_ref[...]
        )


def _node_global(nodes, agg2, wn1, wn2, cn, wg1, wg2, cg):
    blk = 1000
    grid = N // blk
    return pl.pallas_call(
        _node_body,
        grid=(grid,),
        in_specs=[
            pl.BlockSpec((blk, DF), lambda i: (i, 0)),
            pl.BlockSpec((blk, H), lambda i: (i, 0)),
            pl.BlockSpec((blk, H), lambda i: (i + grid, 0)),
            pl.BlockSpec((DF, H), lambda i: (0, 0)),
            pl.BlockSpec((H, H), lambda i: (0, 0)),
            pl.BlockSpec((1, H), lambda i: (0, 0)),
            pl.BlockSpec((H, DG), lambda i: (0, 0)),
            pl.BlockSpec((H, DG), lambda i: (0, 0)),
            pl.BlockSpec((1, DG), lambda i: (0, 0)),
        ],
        out_specs=[
            pl.BlockSpec((blk, H), lambda i: (i, 0)),
            pl.BlockSpec((1, DG), lambda i: (0, 0)),
        ],
        out_shape=[
            jax.ShapeDtypeStruct((N, H), F32),
            jax.ShapeDtypeStruct((1, DG), F32),
        ],
        scratch_shapes=[
            pltpu.VMEM((1, H), F32),
            pltpu.VMEM((1, H), F32),
        ],
    )(nodes, agg2, agg2, wn1, wn2, cn, wg1, wg2, cg)


def kernel(nodes, edges, globals_, We_w, We_b, Wn_w, Wn_b, Wg_w, Wg_b,
           senders, receivers):
    senders = senders.astype(jnp.int32)
    receivers = receivers.astype(jnp.int32)

    we1 = We_w[:DE]
    ws = We_w[DE:DE + DF]
    wd = We_w[DE + DF:DE + 2 * DF]
    ce = (We_b + globals_ @ We_w[DE + 2 * DF:]).reshape(1, H)

    wn1 = Wn_w[:DF]
    wn2 = Wn_w[DF:DF + H]
    cn = (Wn_b + globals_ @ Wn_w[DF + H:]).reshape(1, H)

    wg1 = Wg_w[:H]
    wg2 = Wg_w[H:2 * H]
    cg = (Wg_b + globals_ @ Wg_w[2 * H:]).reshape(1, DG)

    p, q = _project_nodes(nodes, ws, wd)
    xe = _project_edges(edges, we1, ce)
    e_new, agg2 = _sc_edge(p, q, xe, senders, receivers)
    n_new, g_new = _node_global(nodes, agg2, wn1, wn2, cn, wg1, wg2, cg)
    return n_new, e_new, g_new
